# baseline (speedup 1.0000x reference)
import functools

import jax
import jax.numpy as jnp
from jax import lax
from jax.experimental import pallas as pl
from jax.experimental.pallas import tpu as pltpu
from jax.experimental.pallas import tpu_sc as plsc
from jax.scipy.special import logsumexp

F = 26
C = 16
CARD = 256
B = 16384
NT = F - 1
NC, NS = 2, 16
NW = NC * NS
B_PER_W = B // NW
CHUNK = 32
NCHUNK = B_PER_W // CHUNK
ROWS = CHUNK * NT


def _lse_body(extra_ref, w_ref, t_ref):
    j = pl.program_id(0)
    w = w_ref[0]
    s = jnp.sum(jnp.exp(w), axis=0)
    t = -jnp.log(s)
    t_ref[0] = jnp.where(j == 0, t + extra_ref[...], t)


def _norm_tables(class_logits, W_self, W_pair):
    cl_norm = class_logits - logsumexp(class_logits)
    ws_norm = W_self - logsumexp(W_self, axis=0)
    extra = (ws_norm + cl_norm[None, :]).T
    wpt = jnp.transpose(W_pair, (0, 1, 3, 2))
    t = pl.pallas_call(
        _lse_body,
        grid=(NT,),
        in_specs=[
            pl.BlockSpec((C, CARD), lambda j: (0, 0)),
            pl.BlockSpec((1, CARD, C, CARD), lambda j: (j, 0, 0, 0)),
        ],
        out_specs=pl.BlockSpec((1, C, CARD), lambda j: (j, 0, 0)),
        out_shape=jax.ShapeDtypeStruct((NT, C, CARD), jnp.float32),
    )(extra, wpt)
    return t.reshape(NT * C, CARD), W_pair.reshape(NT * CARD * CARD, C)


def _gather_sum_body(idxb_hbm, idxs_hbm, wp_hbm, tcp_hbm, out_hbm,
                     idxb_v, idxs_v, rows_b, out_v, slab_v,
                     t_loc, semb):
    cid = lax.axis_index("c")
    sid = lax.axis_index("s")
    wid = sid * NC + cid
    lanes = lax.iota(jnp.int32, C)

    def _xpose(j, _):
        pltpu.sync_copy(tcp_hbm.at[pl.ds(j * C, C)], slab_v)

        def body(p, _):
            t_loc[j * CARD + p] = plsc.load_gather(
                slab_v, [lanes, jnp.full((C,), p, jnp.int32)])
            return 0
        lax.fori_loop(0, CARD, body, 0)
        return 0
    lax.fori_loop(0, NT, _xpose, 0)

    def chunk_body(ch, _):
        g = wid * NCHUNK + ch
        pltpu.sync_copy(idxb_hbm.at[pl.ds(g * ROWS, ROWS)], idxb_v)
        pltpu.sync_copy(idxs_hbm.at[pl.ds(g * CHUNK * 32, CHUNK * 32)],
                        idxs_v)
        d = pltpu.make_async_copy(wp_hbm.at[idxb_v], rows_b, semb)
        d.start()

        def tbody(bl, _):
            v0 = idxs_v[pl.ds(bl * 32, C)]
            v1 = idxs_v[pl.ds(bl * 32 + C, C)]
            acc = plsc.load_gather(
                t_loc, [jnp.full((C,), v0[0], jnp.int32), lanes])
            for j in range(1, NT):
                r = v0[j] if j < C else v1[j - C]
                acc = acc + plsc.load_gather(
                    t_loc, [jnp.full((C,), r, jnp.int32), lanes])
            out_v[bl] = acc
            return 0
        lax.fori_loop(0, CHUNK, tbody, 0)
        d.wait()

        def bbody(bl, _):
            acc = out_v[bl] + rows_b[bl]
            for j in range(1, NT):
                acc = acc + rows_b[j * CHUNK + bl]
            out_v[bl] = acc
            return 0
        lax.fori_loop(0, CHUNK, bbody, 0)
        pltpu.sync_copy(out_v, out_hbm.at[pl.ds(wid * B_PER_W + ch * CHUNK,
                                                CHUNK)])
        return 0
    lax.fori_loop(0, NCHUNK, chunk_body, 0)


def kernel(x, class_logits, W_self, W_pair, training):
    del training
    xi = x.astype(jnp.int32)
    t2, wp2 = _norm_tables(class_logits, W_self, W_pair)
    j_ar = jnp.arange(NT, dtype=jnp.int32)[None, :]
    nblk = B // CHUNK
    idx_big = (j_ar * (CARD * CARD) + xi[:, 1:] * CARD + xi[:, :-1])
    idx_small = (j_ar * CARD + xi[:, :NT])
    idx_big = idx_big.reshape(nblk, CHUNK, NT).transpose(0, 2, 1).reshape(-1)
    idx_small = jnp.pad(idx_small, ((0, 0), (0, 32 - NT))).reshape(-1)

    mesh = plsc.VectorSubcoreMesh(core_axis_name="c", subcore_axis_name="s",
                                  num_cores=NC, num_subcores=NS)
    run = functools.partial(
        pl.kernel,
        out_type=jax.ShapeDtypeStruct((B, C), jnp.float32),
        mesh=mesh,
        compiler_params=pltpu.CompilerParams(use_tc_tiling_on_sc=False,
                                             needs_layout_passes=False),
        scratch_types=[
            pltpu.VMEM((ROWS,), jnp.int32),
            pltpu.VMEM((CHUNK * 32,), jnp.int32),
            pltpu.VMEM((ROWS, C), jnp.float32),
            pltpu.VMEM((CHUNK, C), jnp.float32),
            pltpu.VMEM((C, CARD), jnp.float32),
            pltpu.VMEM((NT * CARD, C), jnp.float32),
            pltpu.SemaphoreType.DMA,
        ],
    )(_gather_sum_body)
    return run(idx_big, idx_small, wp2, t2)
